# baseline (speedup 1.0000x reference)
import functools

import jax
import jax.numpy as jnp
from jax import lax
from jax.experimental import pallas as pl
from jax.experimental.pallas import tpu as pltpu
from jax.experimental.pallas import tpu_sc as plsc

_B, _L, _D, _E = 4096, 50, 20, 32
_NW = 32
_BPW = _B // _NW
_GSZ = 128
_G = 4


def _sc_body(syntax_hbm, table_hbm, pos_hbm, out_hbm,
             idx_a, idx_b, slab_a, slab_b, acc_a, acc_b, pos_v,
             isem_a, isem_b, gsem_a, gsem_b, osem_a, osem_b):
    wid = lax.axis_index("s") * 2 + lax.axis_index("c")
    b0 = wid * _BPW
    pltpu.sync_copy(pos_hbm, pos_v)

    def idx_start(l, idx_v, isem):
        pltpu.async_copy(syntax_hbm.at[:, l, pl.ds(b0, _BPW)], idx_v, isem)

    def idx_wait(idx_v, isem):
        pltpu.make_async_copy(
            syntax_hbm.at[:, 0, pl.ds(b0, _BPW)], idx_v, isem).wait()

    def fire(idx_v, slab, gsem):
        for d in range(_D):
            pltpu.async_copy(
                table_hbm.at[idx_v.at[d]],
                slab.at[pl.ds(d * _GSZ, _GSZ)],
                gsem,
            )

    def drain(idx_v, slab, gsem):
        for d in range(_D):
            pltpu.make_async_copy(
                table_hbm.at[idx_v.at[d]],
                slab.at[pl.ds(d * _GSZ, _GSZ)],
                gsem,
            ).wait()

    e_even = lax.iota(jnp.int32, 16) * 2
    e_odd = e_even + 1

    def compute(slab, acc):
        @plsc.parallel_loop(0, _BPW // _G, 1, unroll=2)
        def _(g):
            racc = [jnp.zeros((32,), jnp.bfloat16) for _ in range(_G)]
            for d in range(_D):
                p = pos_v[d, pl.ds(0, _E)]
                base = d * _GSZ + g * _G
                for r in range(_G):
                    row = plsc.bitcast(slab[base + r, pl.ds(0, _E // 2)],
                                       jnp.bfloat16)
                    racc[r] += row * p
            for r in range(_G):
                j = jnp.full((16,), g * _G + r, jnp.int32)
                v_even, v_odd = plsc.unpack(racc[r],
                                            format=plsc.PackFormat.INTERLEAVED)
                plsc.store_scatter(acc, [e_even, j], v_even)
                plsc.store_scatter(acc, [e_odd, j], v_odd)

    def out_wait(acc, osem):
        for k in range(4):
            pltpu.make_async_copy(
                acc.at[pl.ds(8 * k, 8)], out_hbm.at[0, k, wid], osem,
            ).wait()

    def out_send(l, acc, osem):
        for k in range(4):
            pltpu.async_copy(
                acc.at[pl.ds(8 * k, 8)], out_hbm.at[l, k, wid], osem,
            )

    idx_start(0, idx_a, isem_a)
    idx_wait(idx_a, isem_a)
    fire(idx_a, slab_a, gsem_a)
    idx_start(1, idx_b, isem_b)

    def pair_body(t, _):
        l0 = 2 * t
        l1 = l0 + 1
        last = t >= _L // 2 - 1

        idx_wait(idx_b, isem_b)
        fire(idx_b, slab_b, gsem_b)
        drain(idx_a, slab_a, gsem_a)

        @pl.when(jnp.logical_not(last))
        def _():
            idx_start(l0 + 2, idx_a, isem_a)

        @pl.when(t >= 1)
        def _():
            out_wait(acc_a, osem_a)

        compute(slab_a, acc_a)
        out_send(l0, acc_a, osem_a)

        @pl.when(jnp.logical_not(last))
        def _():
            idx_wait(idx_a, isem_a)
            fire(idx_a, slab_a, gsem_a)

        drain(idx_b, slab_b, gsem_b)

        @pl.when(t >= 1)
        def _():
            out_wait(acc_b, osem_b)

        compute(slab_b, acc_b)
        out_send(l1, acc_b, osem_b)

        @pl.when(jnp.logical_not(last))
        def _():
            idx_start(l1 + 2, idx_b, isem_b)

        return 0

    lax.fori_loop(0, _L // 2, pair_body, 0, unroll=False)
    out_wait(acc_a, osem_a)
    out_wait(acc_b, osem_b)


@jax.jit
def _syntax_embed(syntax_dlb, emb_table, pos_emb):
    mesh = plsc.VectorSubcoreMesh(core_axis_name="c", subcore_axis_name="s")
    return pl.kernel(
        _sc_body,
        out_type=jax.ShapeDtypeStruct((_L, 4, _NW, 8, 128), jnp.float32),
        mesh=mesh,
        compiler_params=pltpu.CompilerParams(use_tc_tiling_on_sc=False,
                                             needs_layout_passes=False),
        scratch_types=[
            pltpu.VMEM((_D, _BPW), jnp.int32),
            pltpu.VMEM((_D, _BPW), jnp.int32),
            pltpu.VMEM((_D * _GSZ, _E // 2), jnp.int32),
            pltpu.VMEM((_D * _GSZ, _E // 2), jnp.int32),
            pltpu.VMEM((_E, _BPW), jnp.float32),
            pltpu.VMEM((_E, _BPW), jnp.float32),
            pltpu.VMEM((_D, _E), jnp.bfloat16),
            pltpu.SemaphoreType.DMA,
            pltpu.SemaphoreType.DMA,
            pltpu.SemaphoreType.DMA,
            pltpu.SemaphoreType.DMA,
            pltpu.SemaphoreType.DMA,
            pltpu.SemaphoreType.DMA,
        ],
    )(syntax_dlb, emb_table, pos_emb)


def kernel(syntax, emb_table, pos_emb):
    syntax_dlb = syntax.transpose(2, 1, 0)
    v = lax.bitcast_convert_type(emb_table, jnp.uint32)
    hi = (v + 0x7FFF + ((v >> 16) & 1)) >> 16
    packed = hi[:, 0::2] | (hi[:, 1::2] << 16)
    table_pk = lax.bitcast_convert_type(packed, jnp.int32)
    out5 = _syntax_embed(syntax_dlb, table_pk,
                         pos_emb.astype(jnp.bfloat16))
    out = out5.transpose(2, 4, 0, 1, 3).reshape(_B, _L, _E)
    return out


# speedup vs baseline: 3.1583x; 3.1583x over previous
import functools

import jax
import jax.numpy as jnp
from jax import lax
from jax.experimental import pallas as pl
from jax.experimental.pallas import tpu as pltpu
from jax.experimental.pallas import tpu_sc as plsc

_B, _L, _D, _E = 4096, 50, 20, 32
_NW = 32
_BPW = _B // _NW
_GSZ = 128
_G = 4


def _sc_body(syntax_hbm, table_hbm, pos_hbm, out_hbm,
             idx_a, idx_b, slab_a, slab_b, acc_a, acc_b, pos_v,
             isem_a, isem_b, gsem_a, gsem_b, osem_a, osem_b):
    wid = lax.axis_index("s") * 2 + lax.axis_index("c")
    b0 = wid * _BPW
    pltpu.sync_copy(pos_hbm, pos_v)

    def idx_start(l, idx_v, isem):
        pltpu.async_copy(syntax_hbm.at[:, l, pl.ds(b0, _BPW)], idx_v, isem)

    def idx_wait(idx_v, isem):
        pltpu.make_async_copy(
            syntax_hbm.at[:, 0, pl.ds(b0, _BPW)], idx_v, isem).wait()

    def fire(idx_v, slab, gsem):
        for d in range(_D):
            pltpu.async_copy(
                table_hbm.at[idx_v.at[d]],
                slab.at[pl.ds(d * _GSZ, _GSZ)],
                gsem,
            )

    def drain(idx_v, slab, gsem):
        for d in range(_D):
            pltpu.make_async_copy(
                table_hbm.at[idx_v.at[d]],
                slab.at[pl.ds(d * _GSZ, _GSZ)],
                gsem,
            ).wait()

    e_even = lax.iota(jnp.int32, 16) * 2
    e_odd = e_even + 1

    def compute(slab, acc):
        @plsc.parallel_loop(0, _BPW // _G, 1, unroll=2)
        def _(g):
            racc = [jnp.zeros((32,), jnp.bfloat16) for _ in range(_G)]
            for d in range(_D):
                p = pos_v[d, pl.ds(0, _E)]
                base = d * _GSZ + g * _G
                for r in range(_G):
                    racc[r] += slab[base + r, pl.ds(0, _E)] * p
            for r in range(_G):
                j = jnp.full((16,), g * _G + r, jnp.int32)
                v_even, v_odd = plsc.unpack(racc[r],
                                            format=plsc.PackFormat.INTERLEAVED)
                plsc.store_scatter(acc, [e_even, j], v_even)
                plsc.store_scatter(acc, [e_odd, j], v_odd)

    def out_wait(acc, osem):
        for k in range(4):
            pltpu.make_async_copy(
                acc.at[pl.ds(8 * k, 8)], out_hbm.at[0, k, wid], osem,
            ).wait()

    def out_send(l, acc, osem):
        for k in range(4):
            pltpu.async_copy(
                acc.at[pl.ds(8 * k, 8)], out_hbm.at[l, k, wid], osem,
            )

    idx_start(0, idx_a, isem_a)
    idx_wait(idx_a, isem_a)
    fire(idx_a, slab_a, gsem_a)
    idx_start(1, idx_b, isem_b)

    def pair_body(t, _):
        l0 = 2 * t
        l1 = l0 + 1
        last = t >= _L // 2 - 1

        idx_wait(idx_b, isem_b)
        fire(idx_b, slab_b, gsem_b)
        drain(idx_a, slab_a, gsem_a)

        @pl.when(jnp.logical_not(last))
        def _():
            idx_start(l0 + 2, idx_a, isem_a)

        @pl.when(t >= 1)
        def _():
            out_wait(acc_a, osem_a)

        compute(slab_a, acc_a)
        out_send(l0, acc_a, osem_a)

        @pl.when(jnp.logical_not(last))
        def _():
            idx_wait(idx_a, isem_a)
            fire(idx_a, slab_a, gsem_a)

        drain(idx_b, slab_b, gsem_b)

        @pl.when(t >= 1)
        def _():
            out_wait(acc_b, osem_b)

        compute(slab_b, acc_b)
        out_send(l1, acc_b, osem_b)

        @pl.when(jnp.logical_not(last))
        def _():
            idx_start(l1 + 2, idx_b, isem_b)

        return 0

    lax.fori_loop(0, _L // 2, pair_body, 0, unroll=False)
    out_wait(acc_a, osem_a)
    out_wait(acc_b, osem_b)


@jax.jit
def _syntax_embed(syntax_dlb, emb_table, pos_emb):
    mesh = plsc.VectorSubcoreMesh(core_axis_name="c", subcore_axis_name="s")
    return pl.kernel(
        _sc_body,
        out_type=jax.ShapeDtypeStruct((_L, 4, _NW, 8, 128), jnp.float32),
        mesh=mesh,
        compiler_params=pltpu.CompilerParams(use_tc_tiling_on_sc=False,
                                             needs_layout_passes=False),
        scratch_types=[
            pltpu.VMEM((_D, _BPW), jnp.int32),
            pltpu.VMEM((_D, _BPW), jnp.int32),
            pltpu.VMEM((_D * _GSZ, _E), jnp.bfloat16),
            pltpu.VMEM((_D * _GSZ, _E), jnp.bfloat16),
            pltpu.VMEM((_E, _BPW), jnp.float32),
            pltpu.VMEM((_E, _BPW), jnp.float32),
            pltpu.VMEM((_D, _E), jnp.bfloat16),
            pltpu.SemaphoreType.DMA,
            pltpu.SemaphoreType.DMA,
            pltpu.SemaphoreType.DMA,
            pltpu.SemaphoreType.DMA,
            pltpu.SemaphoreType.DMA,
            pltpu.SemaphoreType.DMA,
        ],
    )(syntax_dlb, emb_table, pos_emb)


def kernel(syntax, emb_table, pos_emb):
    syntax_dlb = syntax.transpose(2, 1, 0)
    table_bf = emb_table.astype(jnp.bfloat16)
    out5 = _syntax_embed(syntax_dlb, table_bf,
                         pos_emb.astype(jnp.bfloat16))
    out = out5.transpose(2, 4, 0, 1, 3).reshape(_B, _L, _E)
    return out
